# baseline (speedup 1.0000x reference)
import functools

import jax
import jax.numpy as jnp
from jax import lax
from jax.experimental import pallas as pl
from jax.experimental.pallas import tpu as pltpu
from jax.experimental.pallas import tpu_sc as plsc

N = 10000
E = 320000
D = 128
NEG = 0.01
EPS = 1e-5

NC = 2
NS = 16
DH = D // NC
EPT = E // NS
CHUNK = 80
NCHUNK = EPT // CHUNK
RPT = (N // NS) & ~7
RTAIL = N - NS * RPT

BLK = 1000
NB = N // BLK


def _leaky(v):
    return jnp.where(v >= 0, v, NEG * v)


def _sc_body(x0_hbm, x1_hbm, src_hbm, dst_hbm, attr_hbm, zeros_hbm, out_hbm,
             src_v, dst_v, attr_v, rows0_v, rows1_v, rows2_v, rows3_v,
             orows0_v, orows1_v, orows2_v, orows3_v,
             acc_sh, gsem0, gsem1, gsem2, gsem3,
             ssem0, ssem1, ssem2, ssem3):
    cid = lax.axis_index("c")
    sid = lax.axis_index("s")

    rsl = pl.ds(sid * RPT, RPT)
    tsl = pl.ds(NS * RPT, RTAIL)
    pltpu.sync_copy(zeros_hbm.at[rsl], acc_sh.at[rsl])

    @pl.when(sid == NS - 1)
    def _zero_tail():
        pltpu.sync_copy(zeros_hbm.at[tsl], acc_sh.at[tsl])

    pltpu.sync_copy(src_hbm.at[sid], src_v)
    pltpu.sync_copy(dst_hbm.at[sid], dst_v)
    pltpu.sync_copy(attr_hbm.at[sid], attr_v)
    plsc.subcore_barrier()

    def edge_loop(xc_hbm):
        bufs = (rows0_v, rows1_v, rows2_v, rows3_v)
        obufs = (orows0_v, orows1_v, orows2_v, orows3_v)
        gsems = (gsem0, gsem1, gsem2, gsem3)
        ssems = (ssem0, ssem1, ssem2, ssem3)
        NBUF = 4

        def gather(c, b):
            pltpu.async_copy(xc_hbm.at[src_v.at[c]], bufs[b], gsems[b])

        def gwait(c, b):
            pltpu.make_async_copy(xc_hbm.at[src_v.at[c]], bufs[b],
                                  gsems[b]).wait()

        def scatter(c, b):
            pltpu.async_copy(obufs[b], acc_sh.at[dst_v.at[c]], ssems[b],
                             add=True)

        def swait(c, b):
            pltpu.make_async_copy(obufs[b], acc_sh.at[dst_v.at[c]],
                                  ssems[b]).wait()

        def scale(c, b):
            rows, orows = bufs[b], obufs[b]

            def scale_body(g, carry2):
                av = attr_v[c, pl.ds(g * 16, 16)]
                for l0 in range(0, 16, 4):
                    aa = [av[l0 + tt] for tt in range(4)]
                    for h in range(DH // 32):
                        for tt in range(4):
                            e = g * 16 + l0 + tt
                            v32 = rows[e, pl.ds(h * 16, 16)]
                            ev = lax.bitcast_convert_type(
                                v32 << 16, jnp.float32)
                            od = lax.bitcast_convert_type(
                                v32 & jnp.int32(-65536), jnp.float32)
                            orows[e, pl.ds(h * 32, 16)] = ev * aa[tt]
                            orows[e, pl.ds(h * 32 + 16, 16)] = od * aa[tt]
                return carry2

            lax.fori_loop(0, CHUNK // 16, scale_body, 0, unroll=False)

        gather(0, 0)
        gather(1, 1)

        def slot(c, swait_c=None, gather_c=None):
            if swait_c is not None:
                swait(swait_c, swait_c % NBUF)
            if gather_c is not None:
                gather(gather_c, gather_c % NBUF)
            gwait(c, c % NBUF)
            scale(c, c % NBUF)
            scatter(c, c % NBUF)

        slot(0, swait_c=None, gather_c=2)
        slot(1, swait_c=None, gather_c=3)
        slot(2, swait_c=0, gather_c=4)
        slot(3, swait_c=1, gather_c=5)

        def quad_body(k, carry):
            c0 = k * NBUF
            for b in range(NBUF):
                c = c0 + b
                swait(c - 2, (b + 2) % NBUF)
                gather(c + 2, (b + 2) % NBUF)
                gwait(c, b)
                scale(c, b)
                scatter(c, b)
            return carry

        lax.fori_loop(1, (NCHUNK - 4 - 6) // NBUF + 1, quad_body, 0,
                      unroll=False)
        base = NCHUNK - 6
        for off in range(6):
            c = base + off
            slot(c, swait_c=c - 2, gather_c=(c + 2 if off < 4 else None))
        swait(NCHUNK - 2, (NCHUNK - 2) % NBUF)
        swait(NCHUNK - 1, (NCHUNK - 1) % NBUF)

    @pl.when(cid == 0)
    def _lo_half():
        edge_loop(x0_hbm)

    @pl.when(cid == 1)
    def _hi_half():
        edge_loop(x1_hbm)

    plsc.subcore_barrier()
    pltpu.sync_copy(acc_sh.at[rsl], out_hbm.at[cid, rsl])

    @pl.when(sid == NS - 1)
    def _write_tail():
        pltpu.sync_copy(acc_sh.at[tsl], out_hbm.at[cid, tsl])


@functools.cache
def _sc_aggregate_fn():
    mesh = plsc.VectorSubcoreMesh(core_axis_name="c", subcore_axis_name="s")
    return pl.kernel(
        _sc_body,
        out_type=jax.ShapeDtypeStruct((NC, N, DH), jnp.float32),
        mesh=mesh,
        scratch_types=[
            pltpu.VMEM((NCHUNK, CHUNK), jnp.int32),
            pltpu.VMEM((NCHUNK, CHUNK), jnp.int32),
            pltpu.VMEM((NCHUNK, CHUNK), jnp.float32),
            pltpu.VMEM((CHUNK, DH // 2), jnp.int32),
            pltpu.VMEM((CHUNK, DH // 2), jnp.int32),
            pltpu.VMEM((CHUNK, DH // 2), jnp.int32),
            pltpu.VMEM((CHUNK, DH // 2), jnp.int32),
            pltpu.VMEM((CHUNK, DH), jnp.float32),
            pltpu.VMEM((CHUNK, DH), jnp.float32),
            pltpu.VMEM((CHUNK, DH), jnp.float32),
            pltpu.VMEM((CHUNK, DH), jnp.float32),
            pltpu.VMEM_SHARED((N, DH), jnp.float32),
            pltpu.SemaphoreType.DMA,
            pltpu.SemaphoreType.DMA,
            pltpu.SemaphoreType.DMA,
            pltpu.SemaphoreType.DMA,
            pltpu.SemaphoreType.DMA,
            pltpu.SemaphoreType.DMA,
            pltpu.SemaphoreType.DMA,
            pltpu.SemaphoreType.DMA,
        ],
        compiler_params=pltpu.CompilerParams(use_tc_tiling_on_sc=False),
    )


def _pack_body(x_ref, o_ref):
    u = lax.bitcast_convert_type(x_ref[...], jnp.int32)
    r = ((u + 0x7FFF + ((u >> 16) & 1)) >> 16) & 0xFFFF
    q = D // 4
    o_ref[0] = r[:, 0 * q:1 * q] | (r[:, 1 * q:2 * q] << 16)
    o_ref[1] = r[:, 2 * q:3 * q] | (r[:, 3 * q:4 * q] << 16)


def _pack_x(x):
    return pl.pallas_call(
        _pack_body,
        grid=(NB,),
        in_specs=[pl.BlockSpec((BLK, D), lambda i: (i, 0))],
        out_specs=pl.BlockSpec((NC, BLK, D // 4), lambda i: (0, i, 0)),
        out_shape=jax.ShapeDtypeStruct((NC, N, D // 4), jnp.int32),
    )(x)


def _tc_body(p_ref, x_ref, wrT_ref, br_ref, wtT_ref, wlT_ref, bl_ref,
             g_ref, b_ref, o_ref, x3_s, sum_s, sq_s):
    p = pl.program_id(0)
    i = pl.program_id(1)

    @pl.when(p == 0)
    def _compute():
        @pl.when(i == 0)
        def _init():
            sum_s[...] = jnp.zeros_like(sum_s)
            sq_s[...] = jnp.zeros_like(sq_s)

        agg = jnp.concatenate([p_ref[0], p_ref[1]], axis=1)
        x1 = (jnp.dot(agg, wrT_ref[...], preferred_element_type=jnp.float32)
              + br_ref[...]
              + jnp.dot(x_ref[...], wtT_ref[...],
                        preferred_element_type=jnp.float32))
        x2 = _leaky(x1)
        x3 = (jnp.dot(x2, wlT_ref[...], preferred_element_type=jnp.float32)
              + bl_ref[...])
        x3_s[pl.ds(i * BLK, BLK), :] = x3
        sum_s[...] += jnp.sum(x3, axis=0, keepdims=True)
        sq_s[...] += jnp.sum(x3 * x3, axis=0, keepdims=True)

    @pl.when(p == 1)
    def _normalize():
        mean = sum_s[...] * (1.0 / N)
        var = sq_s[...] * (1.0 / N) - mean * mean
        inv = lax.rsqrt(var + EPS)
        x3 = x3_s[pl.ds(i * BLK, BLK), :]
        x4 = (x3 - mean) * (inv * g_ref[...]) + b_ref[...]
        o_ref[...] = _leaky(x4)


def _tc_dense(partials, x, wrT, br, wtT, wlT, bl, g, b):
    full = lambda p, i: (0, 0)
    return pl.pallas_call(
        _tc_body,
        grid=(2, NB),
        in_specs=[
            pl.BlockSpec((NC, BLK, DH),
                         lambda p, i: (0, jnp.where(p == 0, i, NB - 1), 0)),
            pl.BlockSpec((BLK, D),
                         lambda p, i: (jnp.where(p == 0, i, NB - 1), 0)),
            pl.BlockSpec((D, D), full),
            pl.BlockSpec((1, D), full),
            pl.BlockSpec((D, D), full),
            pl.BlockSpec((D, D), full),
            pl.BlockSpec((1, D), full),
            pl.BlockSpec((1, D), full),
            pl.BlockSpec((1, D), full),
        ],
        out_specs=pl.BlockSpec((BLK, D),
                               lambda p, i: (jnp.where(p == 1, i, 0), 0)),
        out_shape=jax.ShapeDtypeStruct((N, D), jnp.float32),
        scratch_shapes=[
            pltpu.VMEM((N, D), jnp.float32),
            pltpu.VMEM((1, D), jnp.float32),
            pltpu.VMEM((1, D), jnp.float32),
        ],
    )(partials, x, wrT, br, wtT, wlT, bl, g, b)


def kernel(x, edge_index, batch, edge_attr, W_rel, b_rel, W_root, W_lin,
           b_lin, gamma, beta):
    del batch
    src = edge_index[0].reshape(NS, NCHUNK, CHUNK)
    dst = edge_index[1].reshape(NS, NCHUNK, CHUNK)
    attr = edge_attr.reshape(NS, NCHUNK, CHUNK)
    zeros = jnp.zeros((N, DH), jnp.float32)
    xw = _pack_x(x)
    partials = _sc_aggregate_fn()(xw[0], xw[1], src, dst, attr, zeros)
    wrT_perm = (W_rel.T.reshape(2, 2, 2, 16, D)
                .transpose(0, 2, 1, 3, 4).reshape(D, D))
    return _tc_dense(partials, x, wrT_perm, b_rel.reshape(1, D), W_root.T,
                     W_lin.T, b_lin.reshape(1, D), gamma.reshape(1, D),
                     beta.reshape(1, D))


# speedup vs baseline: 1.7321x; 1.7321x over previous
import functools

import jax
import jax.numpy as jnp
from jax import lax
from jax.experimental import pallas as pl
from jax.experimental.pallas import tpu as pltpu
from jax.experimental.pallas import tpu_sc as plsc

N = 10000
E = 320000
D = 128
NEG = 0.01
EPS = 1e-5

NC = 2
NS = 16
DH = D // NC
EPT = E // NS
CHUNK = 80
NCHUNK = EPT // CHUNK
RPT = (N // NS) & ~7
RTAIL = N - NS * RPT

BLK = 1000
NB = N // BLK


def _leaky(v):
    return jnp.where(v >= 0, v, NEG * v)


def _sc_body(x0_hbm, x1_hbm, src_hbm, dst_hbm, attr_hbm, zeros_hbm, out_hbm,
             src_v, dst_v, attr_v, rows0_v, rows1_v, rows2_v, rows3_v,
             rows4_v, acc_sh, gsem0, gsem1, gsem2, gsem3, gsem4,
             ssem0, ssem1, ssem2, ssem3, ssem4):
    cid = lax.axis_index("c")
    sid = lax.axis_index("s")

    rsl = pl.ds(sid * RPT, RPT)
    tsl = pl.ds(NS * RPT, RTAIL)
    pltpu.sync_copy(zeros_hbm.at[rsl], acc_sh.at[rsl])

    @pl.when(sid == NS - 1)
    def _zero_tail():
        pltpu.sync_copy(zeros_hbm.at[tsl], acc_sh.at[tsl])

    pltpu.sync_copy(src_hbm.at[sid], src_v)
    pltpu.sync_copy(dst_hbm.at[sid], dst_v)
    pltpu.sync_copy(attr_hbm.at[sid], attr_v)
    plsc.subcore_barrier()

    def edge_loop(xc_hbm):
        bufs = (rows0_v, rows1_v, rows2_v, rows3_v, rows4_v)
        gsems = (gsem0, gsem1, gsem2, gsem3, gsem4)
        ssems = (ssem0, ssem1, ssem2, ssem3, ssem4)
        NBUF = 5

        def gather(c, b):
            pltpu.async_copy(xc_hbm.at[src_v.at[c]], bufs[b], gsems[b])

        def gwait(c, b):
            pltpu.make_async_copy(xc_hbm.at[src_v.at[c]], bufs[b],
                                  gsems[b]).wait()

        def scatter(c, b):
            pltpu.async_copy(bufs[b], acc_sh.at[dst_v.at[c]], ssems[b],
                             add=True)

        def swait(c, b):
            pltpu.make_async_copy(bufs[b], acc_sh.at[dst_v.at[c]],
                                  ssems[b]).wait()

        def scale(c, rows):
            def scale_body(g, carry2):
                av = attr_v[c, pl.ds(g * 16, 16)]
                for l0 in range(0, 16, 4):
                    aa = [av[l0 + tt] for tt in range(4)]
                    for j in range(DH // 16):
                        for tt in range(4):
                            e = g * 16 + l0 + tt
                            sl = pl.ds(j * 16, 16)
                            rows[e, sl] = rows[e, sl] * aa[tt]
                return carry2

            lax.fori_loop(0, CHUNK // 16, scale_body, 0, unroll=False)

        def slot(c, swait_c=None, gather_c=None):
            if swait_c is not None:
                swait(swait_c, swait_c % NBUF)
            if gather_c is not None:
                gather(gather_c, gather_c % NBUF)
            gwait(c, c % NBUF)
            scale(c, bufs[c % NBUF])
            scatter(c, c % NBUF)

        gather(0, 0)
        gather(1, 1)
        slot(0, swait_c=None, gather_c=2)
        slot(1, swait_c=None, gather_c=3)
        slot(2, swait_c=0, gather_c=4)
        slot(3, swait_c=1, gather_c=5)
        slot(4, swait_c=2, gather_c=6)

        def penta_body(k, carry):
            c0 = k * NBUF
            for b in range(NBUF):
                c = c0 + b
                swait(c - 2, (b + 3) % NBUF)
                gather(c + 2, (b + 2) % NBUF)
                gwait(c, b)
                scale(c, bufs[b])
                scatter(c, b)
            return carry

        lax.fori_loop(1, NCHUNK // NBUF - 1, penta_body, 0, unroll=False)
        slot(245, swait_c=243, gather_c=247)
        slot(246, swait_c=244, gather_c=248)
        slot(247, swait_c=245, gather_c=249)
        slot(248, swait_c=246, gather_c=None)
        slot(249, swait_c=247, gather_c=None)
        swait(248, 248 % NBUF)
        swait(249, 249 % NBUF)

    @pl.when(cid == 0)
    def _lo_half():
        edge_loop(x0_hbm)

    @pl.when(cid == 1)
    def _hi_half():
        edge_loop(x1_hbm)

    plsc.subcore_barrier()
    pltpu.sync_copy(acc_sh.at[rsl], out_hbm.at[cid, rsl])

    @pl.when(sid == NS - 1)
    def _write_tail():
        pltpu.sync_copy(acc_sh.at[tsl], out_hbm.at[cid, tsl])


@functools.cache
def _sc_aggregate_fn():
    mesh = plsc.VectorSubcoreMesh(core_axis_name="c", subcore_axis_name="s")
    return pl.kernel(
        _sc_body,
        out_type=jax.ShapeDtypeStruct((NC, N, DH), jnp.float32),
        mesh=mesh,
        scratch_types=[
            pltpu.VMEM((NCHUNK, CHUNK), jnp.int32),
            pltpu.VMEM((NCHUNK, CHUNK), jnp.int32),
            pltpu.VMEM((NCHUNK, CHUNK), jnp.float32),
            pltpu.VMEM((CHUNK, DH), jnp.float32),
            pltpu.VMEM((CHUNK, DH), jnp.float32),
            pltpu.VMEM((CHUNK, DH), jnp.float32),
            pltpu.VMEM((CHUNK, DH), jnp.float32),
            pltpu.VMEM((CHUNK, DH), jnp.float32),
            pltpu.VMEM_SHARED((N, DH), jnp.float32),
            pltpu.SemaphoreType.DMA,
            pltpu.SemaphoreType.DMA,
            pltpu.SemaphoreType.DMA,
            pltpu.SemaphoreType.DMA,
            pltpu.SemaphoreType.DMA,
            pltpu.SemaphoreType.DMA,
            pltpu.SemaphoreType.DMA,
            pltpu.SemaphoreType.DMA,
            pltpu.SemaphoreType.DMA,
            pltpu.SemaphoreType.DMA,
        ],
        compiler_params=pltpu.CompilerParams(use_tc_tiling_on_sc=False),
    )


def _tc_body(p_ref, x_ref, wrT_ref, br_ref, wtT_ref, wlT_ref, bl_ref,
             g_ref, b_ref, o_ref, x3_s, sum_s, sq_s):
    p = pl.program_id(0)
    i = pl.program_id(1)

    @pl.when(p == 0)
    def _compute():
        @pl.when(i == 0)
        def _init():
            sum_s[...] = jnp.zeros_like(sum_s)
            sq_s[...] = jnp.zeros_like(sq_s)

        agg = jnp.concatenate([p_ref[0], p_ref[1]], axis=1)
        x1 = (jnp.dot(agg, wrT_ref[...], preferred_element_type=jnp.float32)
              + br_ref[...]
              + jnp.dot(x_ref[...], wtT_ref[...],
                        preferred_element_type=jnp.float32))
        x2 = _leaky(x1)
        x3 = (jnp.dot(x2, wlT_ref[...], preferred_element_type=jnp.float32)
              + bl_ref[...])
        x3_s[pl.ds(i * BLK, BLK), :] = x3
        sum_s[...] += jnp.sum(x3, axis=0, keepdims=True)
        sq_s[...] += jnp.sum(x3 * x3, axis=0, keepdims=True)

    @pl.when(p == 1)
    def _normalize():
        mean = sum_s[...] * (1.0 / N)
        var = sq_s[...] * (1.0 / N) - mean * mean
        inv = lax.rsqrt(var + EPS)
        x3 = x3_s[pl.ds(i * BLK, BLK), :]
        x4 = (x3 - mean) * (inv * g_ref[...]) + b_ref[...]
        o_ref[...] = _leaky(x4)


def _tc_dense(partials, x, wrT, br, wtT, wlT, bl, g, b):
    full = lambda p, i: (0, 0)
    return pl.pallas_call(
        _tc_body,
        grid=(2, NB),
        in_specs=[
            pl.BlockSpec((NC, BLK, DH),
                         lambda p, i: (0, jnp.where(p == 0, i, NB - 1), 0)),
            pl.BlockSpec((BLK, D),
                         lambda p, i: (jnp.where(p == 0, i, NB - 1), 0)),
            pl.BlockSpec((D, D), full),
            pl.BlockSpec((1, D), full),
            pl.BlockSpec((D, D), full),
            pl.BlockSpec((D, D), full),
            pl.BlockSpec((1, D), full),
            pl.BlockSpec((1, D), full),
            pl.BlockSpec((1, D), full),
        ],
        out_specs=pl.BlockSpec((BLK, D),
                               lambda p, i: (jnp.where(p == 1, i, 0), 0)),
        out_shape=jax.ShapeDtypeStruct((N, D), jnp.float32),
        scratch_shapes=[
            pltpu.VMEM((N, D), jnp.float32),
            pltpu.VMEM((1, D), jnp.float32),
            pltpu.VMEM((1, D), jnp.float32),
        ],
    )(partials, x, wrT, br, wtT, wlT, bl, g, b)


def kernel(x, edge_index, batch, edge_attr, W_rel, b_rel, W_root, W_lin,
           b_lin, gamma, beta):
    del batch
    src = edge_index[0].reshape(NS, NCHUNK, CHUNK)
    dst = edge_index[1].reshape(NS, NCHUNK, CHUNK)
    attr = edge_attr.reshape(NS, NCHUNK, CHUNK)
    zeros = jnp.zeros((N, DH), jnp.float32)
    partials = _sc_aggregate_fn()(x[:, :DH], x[:, DH:], src, dst, attr, zeros)
    return _tc_dense(partials, x, W_rel.T, b_rel.reshape(1, D), W_root.T,
                     W_lin.T, b_lin.reshape(1, D), gamma.reshape(1, D),
                     beta.reshape(1, D))
